# Initial kernel scaffold; baseline (speedup 1.0000x reference)
import jax
import jax.numpy as jnp
from jax.experimental import pallas as pl


def kernel(atom_fea, nbr_fea, nbr_fea_idx, crystal_atom_idx, We, be, Wf0, bf0, g1_0, beta1_0, g2_0, beta2_0, Wf1, bf1, g1_1, beta1_1, g2_1, beta2_1, Wf2, bf2, g1_2, beta1_2, g2_2, beta2_2, Wc, bc, Wo, bo):
    raise NotImplementedError("write your pallas kernel here")


# speedup vs baseline: 1.9216x; 1.9216x over previous
import functools

import jax
import jax.numpy as jnp
from jax import lax
from jax.experimental import pallas as pl
from jax.experimental.pallas import tpu as pltpu
from jax.experimental.pallas import tpu_sc as plsc

_EPS = 1e-5


def _softplus(x):
    return jnp.maximum(x, 0.0) + jnp.log1p(jnp.exp(-jnp.abs(x)))


def _sigmoid(x):
    return 1.0 / (1.0 + jnp.exp(-x))


def _gather_rows(table, idx):
    (b,) = idx.shape
    _, d = table.shape
    info = plsc.get_sparse_core_info()
    nw = info.num_cores * info.num_subcores
    bw = b // nw
    assert bw * nw == b
    chunk = 1000
    while bw % chunk or chunk % 8:
        chunk //= 2
    nch = bw // chunk
    mesh = plsc.VectorSubcoreMesh(core_axis_name="c", subcore_axis_name="s")

    @functools.partial(
        pl.kernel,
        mesh=mesh,
        compiler_params=pltpu.CompilerParams(use_tc_tiling_on_sc=False),
        out_type=jax.ShapeDtypeStruct((b, d), jnp.float32),
        scratch_types=[
            pltpu.VMEM((chunk,), jnp.int32),
            pltpu.VMEM((chunk, d), jnp.float32),
            pltpu.SemaphoreType.DMA,
        ],
    )
    def gk(idx_hbm, tab_hbm, out_hbm, idx_v, rows_v, sem):
        wid = lax.axis_index("s") * info.num_cores + lax.axis_index("c")
        base = wid * bw
        for k in range(nch):
            off = base + k * chunk
            pltpu.sync_copy(idx_hbm.at[pl.ds(off, chunk)], idx_v)
            pltpu.async_copy(tab_hbm.at[idx_v], rows_v, sem).wait()
            pltpu.sync_copy(rows_v, out_hbm.at[pl.ds(off, chunk)])

    return gk(idx, table)


def _embed(atom_fea, we_t, be2):
    n, orig = atom_fea.shape
    af = we_t.shape[1]
    bn = 2000

    def body(a_ref, w_ref, b_ref, o_ref):
        o_ref[...] = (
            jnp.dot(a_ref[...], w_ref[...], preferred_element_type=jnp.float32)
            + b_ref[...]
        )

    return pl.pallas_call(
        body,
        grid=(n // bn,),
        in_specs=[
            pl.BlockSpec((bn, orig), lambda i: (i, 0)),
            pl.BlockSpec((orig, af), lambda i: (0, 0)),
            pl.BlockSpec((1, af), lambda i: (0, 0)),
        ],
        out_specs=pl.BlockSpec((bn, af), lambda i: (i, 0)),
        out_shape=jax.ShapeDtypeStruct((n, af), jnp.float32),
    )(atom_fea, we_t, be2)


def _conv_stats(x, gx, nf2, ws_t, wn_t, we_t, bf2):
    n, af = x.shape
    m = gx.shape[0] // n
    c = ws_t.shape[1]
    nbr = nf2.shape[1]
    bn = 1000

    def body(x_ref, gx_ref, nf_ref, ws_ref, wn_ref, we_ref, bf_ref, out_ref):
        p = jnp.dot(x_ref[...], ws_ref[...], preferred_element_type=jnp.float32)
        e = jnp.dot(gx_ref[...], wn_ref[...], preferred_element_type=jnp.float32)
        e = e + jnp.dot(nf_ref[...], we_ref[...], preferred_element_type=jnp.float32)
        tg = e.reshape(bn, m, c) + p[:, None, :] + bf_ref[...].reshape(1, 1, c)
        s = jnp.sum(tg, axis=(0, 1)).reshape(1, c)
        q = jnp.sum(tg * tg, axis=(0, 1)).reshape(1, c)
        part = jnp.concatenate([s, q], axis=0)

        @pl.when(pl.program_id(0) == 0)
        def _init():
            out_ref[...] = jnp.zeros_like(out_ref)

        out_ref[...] += part

    return pl.pallas_call(
        body,
        grid=(n // bn,),
        in_specs=[
            pl.BlockSpec((bn, af), lambda i: (i, 0)),
            pl.BlockSpec((bn * m, af), lambda i: (i, 0)),
            pl.BlockSpec((bn * m, nbr), lambda i: (i, 0)),
            pl.BlockSpec((af, c), lambda i: (0, 0)),
            pl.BlockSpec((af, c), lambda i: (0, 0)),
            pl.BlockSpec((nbr, c), lambda i: (0, 0)),
            pl.BlockSpec((1, c), lambda i: (0, 0)),
        ],
        out_specs=pl.BlockSpec((2, c), lambda i: (0, 0)),
        out_shape=jax.ShapeDtypeStruct((2, c), jnp.float32),
    )(x, gx, nf2, ws_t, wn_t, we_t, bf2)


def _conv_apply(x, gx, nf2, wts, stats, g1_2, b1_2, nm_total):
    n, af = x.shape
    m = gx.shape[0] // n
    nbr = nf2.shape[1]
    bn = 1000
    inv_nm = 1.0 / nm_total

    def body(
        x_ref, gx_ref, nf_ref,
        wsf_ref, wsc_ref, wnf_ref, wnc_ref, wef_ref, wec_ref, bff_ref, bfc_ref,
        st_ref, g1_ref, b1_ref,
        ns_ref, st2_ref,
    ):
        mean = st_ref[0:1, :] * inv_nm
        var = st_ref[1:2, :] * inv_nm - mean * mean
        scale = g1_ref[...] * lax.rsqrt(var + _EPS)
        shift = b1_ref[...] - mean * scale
        scale_f, scale_c = scale[:, :af], scale[:, af:]
        shift_f, shift_c = shift[:, :af], shift[:, af:]

        pf = jnp.dot(x_ref[...], wsf_ref[...], preferred_element_type=jnp.float32)
        pc = jnp.dot(x_ref[...], wsc_ref[...], preferred_element_type=jnp.float32)
        ef = jnp.dot(gx_ref[...], wnf_ref[...], preferred_element_type=jnp.float32)
        ef = ef + jnp.dot(nf_ref[...], wef_ref[...], preferred_element_type=jnp.float32)
        ec = jnp.dot(gx_ref[...], wnc_ref[...], preferred_element_type=jnp.float32)
        ec = ec + jnp.dot(nf_ref[...], wec_ref[...], preferred_element_type=jnp.float32)

        tgf = ef.reshape(bn, m, af) + (pf + bff_ref[...])[:, None, :]
        tgc = ec.reshape(bn, m, af) + (pc + bfc_ref[...])[:, None, :]
        filt = _sigmoid(tgf * scale_f[:, None, :] + shift_f[:, None, :])
        core = _softplus(tgc * scale_c[:, None, :] + shift_c[:, None, :])
        summed = jnp.sum(filt * core, axis=1)
        ns_ref[...] = summed

        s = jnp.sum(summed, axis=0).reshape(1, af)
        q = jnp.sum(summed * summed, axis=0).reshape(1, af)

        @pl.when(pl.program_id(0) == 0)
        def _init():
            st2_ref[...] = jnp.zeros_like(st2_ref)

        st2_ref[...] += jnp.concatenate([s, q], axis=0)

    wsf, wsc, wnf, wnc, wef, wec, bff, bfc = wts
    c2 = 2 * af
    return pl.pallas_call(
        body,
        grid=(n // bn,),
        in_specs=[
            pl.BlockSpec((bn, af), lambda i: (i, 0)),
            pl.BlockSpec((bn * m, af), lambda i: (i, 0)),
            pl.BlockSpec((bn * m, nbr), lambda i: (i, 0)),
            pl.BlockSpec((af, af), lambda i: (0, 0)),
            pl.BlockSpec((af, af), lambda i: (0, 0)),
            pl.BlockSpec((af, af), lambda i: (0, 0)),
            pl.BlockSpec((af, af), lambda i: (0, 0)),
            pl.BlockSpec((nbr, af), lambda i: (0, 0)),
            pl.BlockSpec((nbr, af), lambda i: (0, 0)),
            pl.BlockSpec((1, af), lambda i: (0, 0)),
            pl.BlockSpec((1, af), lambda i: (0, 0)),
            pl.BlockSpec((2, c2), lambda i: (0, 0)),
            pl.BlockSpec((1, c2), lambda i: (0, 0)),
            pl.BlockSpec((1, c2), lambda i: (0, 0)),
        ],
        out_specs=[
            pl.BlockSpec((bn, af), lambda i: (i, 0)),
            pl.BlockSpec((2, af), lambda i: (0, 0)),
        ],
        out_shape=[
            jax.ShapeDtypeStruct((n, af), jnp.float32),
            jax.ShapeDtypeStruct((2, af), jnp.float32),
        ],
    )(x, gx, nf2, wsf, wsc, wnf, wnc, wef, wec, bff, bfc, stats, g1_2, b1_2)


def _bn2_residual(x, ns, st2, g2_2, b2_2):
    n, af = x.shape
    bn = 2000
    inv_n = 1.0 / n

    def body(x_ref, ns_ref, st_ref, g_ref, b_ref, o_ref):
        mean = st_ref[0:1, :] * inv_n
        var = st_ref[1:2, :] * inv_n - mean * mean
        scale = g_ref[...] * lax.rsqrt(var + _EPS)
        shift = b_ref[...] - mean * scale
        o_ref[...] = _softplus(x_ref[...] + ns_ref[...] * scale + shift)

    return pl.pallas_call(
        body,
        grid=(n // bn,),
        in_specs=[
            pl.BlockSpec((bn, af), lambda i: (i, 0)),
            pl.BlockSpec((bn, af), lambda i: (i, 0)),
            pl.BlockSpec((2, af), lambda i: (0, 0)),
            pl.BlockSpec((1, af), lambda i: (0, 0)),
            pl.BlockSpec((1, af), lambda i: (0, 0)),
        ],
        out_specs=pl.BlockSpec((bn, af), lambda i: (i, 0)),
        out_shape=jax.ShapeDtypeStruct((n, af), jnp.float32),
    )(x, ns, st2, g2_2, b2_2)


def _bn2_residual_head(x, ns, st2, g2_2, b2_2, wc_t, bc2, wo_t, bo2, n0, a):
    n, af = x.shape
    h = wc_t.shape[1]
    inv_n = 1.0 / n
    inv_a = 1.0 / a

    def body(x_ref, ns_ref, st_ref, g_ref, b_ref, wc_ref, bc_ref, wo_ref, bo_ref, o_ref):
        mean = st_ref[0:1, :] * inv_n
        var = st_ref[1:2, :] * inv_n - mean * mean
        scale = g_ref[...] * lax.rsqrt(var + _EPS)
        shift = b_ref[...] - mean * scale
        xn = _softplus(x_ref[...] + ns_ref[...] * scale + shift)
        crys = jnp.sum(xn.reshape(n0, a, af), axis=1) * inv_a
        t = _softplus(crys)
        t = jnp.dot(t, wc_ref[...], preferred_element_type=jnp.float32) + bc_ref[...]
        t = _softplus(t)
        o_ref[...] = (
            jnp.dot(t, wo_ref[...], preferred_element_type=jnp.float32) + bo_ref[...]
        )

    return pl.pallas_call(
        body,
        grid=(1,),
        in_specs=[
            pl.BlockSpec((n, af), lambda i: (0, 0)),
            pl.BlockSpec((n, af), lambda i: (0, 0)),
            pl.BlockSpec((2, af), lambda i: (0, 0)),
            pl.BlockSpec((1, af), lambda i: (0, 0)),
            pl.BlockSpec((1, af), lambda i: (0, 0)),
            pl.BlockSpec((af, h), lambda i: (0, 0)),
            pl.BlockSpec((1, h), lambda i: (0, 0)),
            pl.BlockSpec((h, 1), lambda i: (0, 0)),
            pl.BlockSpec((1, 1), lambda i: (0, 0)),
        ],
        out_specs=pl.BlockSpec((n0, 1), lambda i: (0, 0)),
        out_shape=jax.ShapeDtypeStruct((n0, 1), jnp.float32),
    )(x, ns, st2, g2_2, b2_2, wc_t, bc2, wo_t, bo2)


def kernel(atom_fea, nbr_fea, nbr_fea_idx, crystal_atom_idx, We, be, Wf0, bf0, g1_0, beta1_0, g2_0, beta2_0, Wf1, bf1, g1_1, beta1_1, g2_1, beta2_1, Wf2, bf2, g1_2, beta1_2, g2_2, beta2_2, Wc, bc, Wo, bo):
    n, m = nbr_fea_idx.shape
    af = We.shape[0]
    nbr = nbr_fea.shape[2]
    n0, a = crystal_atom_idx.shape

    nf2 = nbr_fea.reshape(n * m, nbr)
    idx_flat = nbr_fea_idx.reshape(-1).astype(jnp.int32)

    x = _embed(atom_fea, We.T, be.reshape(1, -1))

    conv_params = (
        (Wf0, bf0, g1_0, beta1_0, g2_0, beta2_0),
        (Wf1, bf1, g1_1, beta1_1, g2_1, beta2_1),
        (Wf2, bf2, g1_2, beta1_2, g2_2, beta2_2),
    )

    for li, (Wf, bf, g1, b1, g2, b2) in enumerate(conv_params):
        ws_t = Wf[:, :af].T
        wn_t = Wf[:, af : 2 * af].T
        we_t = Wf[:, 2 * af :].T
        wts = (
            ws_t[:, :af], ws_t[:, af:],
            wn_t[:, :af], wn_t[:, af:],
            we_t[:, :af], we_t[:, af:],
            bf[:af].reshape(1, -1), bf[af:].reshape(1, -1),
        )
        gx = _gather_rows(x, idx_flat)
        stats = _conv_stats(x, gx, nf2, ws_t, wn_t, we_t, bf.reshape(1, -1))
        ns, st2 = _conv_apply(
            x, gx, nf2, wts, stats, g1.reshape(1, -1), b1.reshape(1, -1), n * m
        )
        if li < 2:
            x = _bn2_residual(x, ns, st2, g2.reshape(1, -1), b2.reshape(1, -1))
        else:
            out = _bn2_residual_head(
                x, ns, st2, g2.reshape(1, -1), b2.reshape(1, -1),
                Wc.T, bc.reshape(1, -1), Wo.T, bo.reshape(1, -1), n0, a,
            )
    return out


# speedup vs baseline: 2.4847x; 1.2930x over previous
import functools

import jax
import jax.numpy as jnp
from jax import lax
from jax.experimental import pallas as pl
from jax.experimental.pallas import tpu as pltpu
from jax.experimental.pallas import tpu_sc as plsc

_EPS = 1e-5


def _softplus(x):
    return jnp.maximum(x, 0.0) + jnp.log1p(jnp.exp(-jnp.abs(x)))


def _sigmoid(x):
    return 1.0 / (1.0 + jnp.exp(-x))


def _gather_rows(table, idx):
    (b,) = idx.shape
    _, d = table.shape
    info = plsc.get_sparse_core_info()
    nw = info.num_cores * info.num_subcores
    bw = b // nw
    assert bw * nw == b
    chunk = 200
    assert bw % chunk == 0 and chunk % 8 == 0
    nch = bw // chunk
    mesh = plsc.VectorSubcoreMesh(core_axis_name="c", subcore_axis_name="s")

    @functools.partial(
        pl.kernel,
        mesh=mesh,
        out_type=jax.ShapeDtypeStruct((b, d), jnp.float32),
        scratch_types=[
            pltpu.VMEM((chunk,), jnp.int32),
            pltpu.VMEM((chunk,), jnp.int32),
            pltpu.VMEM((chunk, d), jnp.float32),
            pltpu.VMEM((chunk, d), jnp.float32),
            pltpu.SemaphoreType.DMA,
            pltpu.SemaphoreType.DMA,
            pltpu.SemaphoreType.DMA,
            pltpu.SemaphoreType.DMA,
        ],
    )
    def gk(idx_hbm, tab_hbm, out_hbm, i0, i1, r0, r1, gs0, gs1, ss0, ss1):
        wid = lax.axis_index("s") * info.num_cores + lax.axis_index("c")
        base = wid * bw
        idx_v, rows, gsem, ssem = [i0, i1], [r0, r1], [gs0, gs1], [ss0, ss1]
        pltpu.sync_copy(idx_hbm.at[pl.ds(base, chunk)], i0)
        gat = [pltpu.async_copy(tab_hbm.at[i0], r0, gs0), None]
        scat = [None, None]
        for k in range(nch):
            cur, nxt = k % 2, (k + 1) % 2
            if k + 1 < nch:
                pltpu.sync_copy(
                    idx_hbm.at[pl.ds(base + (k + 1) * chunk, chunk)], idx_v[nxt]
                )
                if scat[nxt] is not None:
                    scat[nxt].wait()
                gat[nxt] = pltpu.async_copy(
                    tab_hbm.at[idx_v[nxt]], rows[nxt], gsem[nxt]
                )
            gat[cur].wait()
            scat[cur] = pltpu.async_copy(
                rows[cur], out_hbm.at[pl.ds(base + k * chunk, chunk)], ssem[cur]
            )
        for s in scat:
            if s is not None:
                s.wait()

    return gk(idx, table)


def _embed(atom_fea, we_t, be2, wn_t):
    n, orig = atom_fea.shape
    af = we_t.shape[1]
    c = wn_t.shape[1]
    bn = 2000

    def body(a_ref, w_ref, b_ref, wn_ref, x_ref, q_ref):
        x = (
            jnp.dot(a_ref[...], w_ref[...], preferred_element_type=jnp.float32)
            + b_ref[...]
        )
        x_ref[...] = x
        q_ref[...] = jnp.dot(x, wn_ref[...], preferred_element_type=jnp.float32)

    return pl.pallas_call(
        body,
        grid=(n // bn,),
        in_specs=[
            pl.BlockSpec((bn, orig), lambda i: (i, 0)),
            pl.BlockSpec((orig, af), lambda i: (0, 0)),
            pl.BlockSpec((1, af), lambda i: (0, 0)),
            pl.BlockSpec((af, c), lambda i: (0, 0)),
        ],
        out_specs=[
            pl.BlockSpec((bn, af), lambda i: (i, 0)),
            pl.BlockSpec((bn, c), lambda i: (i, 0)),
        ],
        out_shape=[
            jax.ShapeDtypeStruct((n, af), jnp.float32),
            jax.ShapeDtypeStruct((n, c), jnp.float32),
        ],
    )(atom_fea, we_t, be2, wn_t)


def _conv_stats(x, qg, nf2, ws_t, we_t, bf2):
    n, af = x.shape
    m = qg.shape[0] // n
    c = ws_t.shape[1]
    nbr = nf2.shape[1]
    bn = 1000

    def body(x_ref, qg_ref, nf_ref, ws_ref, we_ref, bf_ref, out_ref):
        p = jnp.dot(x_ref[...], ws_ref[...], preferred_element_type=jnp.float32)
        r = jnp.dot(nf_ref[...], we_ref[...], preferred_element_type=jnp.float32)
        tg = (r + qg_ref[...]).reshape(bn, m, c) + (p + bf_ref[...])[:, None, :]
        s = jnp.sum(tg, axis=(0, 1)).reshape(1, c)
        q = jnp.sum(tg * tg, axis=(0, 1)).reshape(1, c)
        part = jnp.concatenate([s, q], axis=0)

        @pl.when(pl.program_id(0) == 0)
        def _init():
            out_ref[...] = jnp.zeros_like(out_ref)

        out_ref[...] += part

    return pl.pallas_call(
        body,
        grid=(n // bn,),
        in_specs=[
            pl.BlockSpec((bn, af), lambda i: (i, 0)),
            pl.BlockSpec((bn * m, c), lambda i: (i, 0)),
            pl.BlockSpec((bn * m, nbr), lambda i: (i, 0)),
            pl.BlockSpec((af, c), lambda i: (0, 0)),
            pl.BlockSpec((nbr, c), lambda i: (0, 0)),
            pl.BlockSpec((1, c), lambda i: (0, 0)),
        ],
        out_specs=pl.BlockSpec((2, c), lambda i: (0, 0)),
        out_shape=jax.ShapeDtypeStruct((2, c), jnp.float32),
    )(x, qg, nf2, ws_t, we_t, bf2)


def _conv_apply(x, qg, nf2, ws_t, we_t, bf2, stats, g1_2, b1_2, nm_total):
    n, af = x.shape
    m = qg.shape[0] // n
    c = ws_t.shape[1]
    nbr = nf2.shape[1]
    bn = 1000
    inv_nm = 1.0 / nm_total

    def body(
        x_ref, qg_ref, nf_ref, ws_ref, we_ref, bf_ref, st_ref, g1_ref, b1_ref,
        ns_ref, st2_ref,
    ):
        mean = st_ref[0:1, :] * inv_nm
        var = st_ref[1:2, :] * inv_nm - mean * mean
        scale = g1_ref[...] * lax.rsqrt(var + _EPS)
        shift = b1_ref[...] - mean * scale

        p = jnp.dot(x_ref[...], ws_ref[...], preferred_element_type=jnp.float32)
        r = jnp.dot(nf_ref[...], we_ref[...], preferred_element_type=jnp.float32)
        tg = (r + qg_ref[...]).reshape(bn, m, c) + (p + bf_ref[...])[:, None, :]
        tgn = tg * scale[:, None, :] + shift[:, None, :]
        filt = _sigmoid(tgn[:, :, :af])
        core = _softplus(tgn[:, :, af:])
        summed = jnp.sum(filt * core, axis=1)
        ns_ref[...] = summed

        s = jnp.sum(summed, axis=0).reshape(1, af)
        q = jnp.sum(summed * summed, axis=0).reshape(1, af)

        @pl.when(pl.program_id(0) == 0)
        def _init():
            st2_ref[...] = jnp.zeros_like(st2_ref)

        st2_ref[...] += jnp.concatenate([s, q], axis=0)

    return pl.pallas_call(
        body,
        grid=(n // bn,),
        in_specs=[
            pl.BlockSpec((bn, af), lambda i: (i, 0)),
            pl.BlockSpec((bn * m, c), lambda i: (i, 0)),
            pl.BlockSpec((bn * m, nbr), lambda i: (i, 0)),
            pl.BlockSpec((af, c), lambda i: (0, 0)),
            pl.BlockSpec((nbr, c), lambda i: (0, 0)),
            pl.BlockSpec((1, c), lambda i: (0, 0)),
            pl.BlockSpec((2, c), lambda i: (0, 0)),
            pl.BlockSpec((1, c), lambda i: (0, 0)),
            pl.BlockSpec((1, c), lambda i: (0, 0)),
        ],
        out_specs=[
            pl.BlockSpec((bn, af), lambda i: (i, 0)),
            pl.BlockSpec((2, af), lambda i: (0, 0)),
        ],
        out_shape=[
            jax.ShapeDtypeStruct((n, af), jnp.float32),
            jax.ShapeDtypeStruct((2, af), jnp.float32),
        ],
    )(x, qg, nf2, ws_t, we_t, bf2, stats, g1_2, b1_2)


def _bn2_residual(x, ns, st2, g2_2, b2_2, wn_t):
    n, af = x.shape
    c = wn_t.shape[1]
    bn = 2000
    inv_n = 1.0 / n

    def body(x_ref, ns_ref, st_ref, g_ref, b_ref, wn_ref, o_ref, q_ref):
        mean = st_ref[0:1, :] * inv_n
        var = st_ref[1:2, :] * inv_n - mean * mean
        scale = g_ref[...] * lax.rsqrt(var + _EPS)
        shift = b_ref[...] - mean * scale
        xn = _softplus(x_ref[...] + ns_ref[...] * scale + shift)
        o_ref[...] = xn
        q_ref[...] = jnp.dot(xn, wn_ref[...], preferred_element_type=jnp.float32)

    return pl.pallas_call(
        body,
        grid=(n // bn,),
        in_specs=[
            pl.BlockSpec((bn, af), lambda i: (i, 0)),
            pl.BlockSpec((bn, af), lambda i: (i, 0)),
            pl.BlockSpec((2, af), lambda i: (0, 0)),
            pl.BlockSpec((1, af), lambda i: (0, 0)),
            pl.BlockSpec((1, af), lambda i: (0, 0)),
            pl.BlockSpec((af, c), lambda i: (0, 0)),
        ],
        out_specs=[
            pl.BlockSpec((bn, af), lambda i: (i, 0)),
            pl.BlockSpec((bn, c), lambda i: (i, 0)),
        ],
        out_shape=[
            jax.ShapeDtypeStruct((n, af), jnp.float32),
            jax.ShapeDtypeStruct((n, c), jnp.float32),
        ],
    )(x, ns, st2, g2_2, b2_2, wn_t)


def _bn2_residual_head(x, ns, st2, g2_2, b2_2, wc_t, bc2, wo_t, bo2, n0, a):
    n, af = x.shape
    h = wc_t.shape[1]
    inv_n = 1.0 / n
    inv_a = 1.0 / a

    def body(x_ref, ns_ref, st_ref, g_ref, b_ref, wc_ref, bc_ref, wo_ref, bo_ref, o_ref):
        mean = st_ref[0:1, :] * inv_n
        var = st_ref[1:2, :] * inv_n - mean * mean
        scale = g_ref[...] * lax.rsqrt(var + _EPS)
        shift = b_ref[...] - mean * scale
        xn = _softplus(x_ref[...] + ns_ref[...] * scale + shift)
        crys = jnp.sum(xn.reshape(n0, a, af), axis=1) * inv_a
        t = _softplus(crys)
        t = jnp.dot(t, wc_ref[...], preferred_element_type=jnp.float32) + bc_ref[...]
        t = _softplus(t)
        o_ref[...] = (
            jnp.dot(t, wo_ref[...], preferred_element_type=jnp.float32) + bo_ref[...]
        )

    return pl.pallas_call(
        body,
        grid=(1,),
        in_specs=[
            pl.BlockSpec((n, af), lambda i: (0, 0)),
            pl.BlockSpec((n, af), lambda i: (0, 0)),
            pl.BlockSpec((2, af), lambda i: (0, 0)),
            pl.BlockSpec((1, af), lambda i: (0, 0)),
            pl.BlockSpec((1, af), lambda i: (0, 0)),
            pl.BlockSpec((af, h), lambda i: (0, 0)),
            pl.BlockSpec((1, h), lambda i: (0, 0)),
            pl.BlockSpec((h, 1), lambda i: (0, 0)),
            pl.BlockSpec((1, 1), lambda i: (0, 0)),
        ],
        out_specs=pl.BlockSpec((n0, 1), lambda i: (0, 0)),
        out_shape=jax.ShapeDtypeStruct((n0, 1), jnp.float32),
    )(x, ns, st2, g2_2, b2_2, wc_t, bc2, wo_t, bo2)


def kernel(atom_fea, nbr_fea, nbr_fea_idx, crystal_atom_idx, We, be, Wf0, bf0, g1_0, beta1_0, g2_0, beta2_0, Wf1, bf1, g1_1, beta1_1, g2_1, beta2_1, Wf2, bf2, g1_2, beta1_2, g2_2, beta2_2, Wc, bc, Wo, bo):
    n, m = nbr_fea_idx.shape
    af = We.shape[0]
    nbr = nbr_fea.shape[2]
    n0, a = crystal_atom_idx.shape

    nf2 = nbr_fea.reshape(n * m, nbr)
    idx_flat = nbr_fea_idx.reshape(-1).astype(jnp.int32)

    conv_params = (
        (Wf0, bf0, g1_0, beta1_0, g2_0, beta2_0),
        (Wf1, bf1, g1_1, beta1_1, g2_1, beta2_1),
        (Wf2, bf2, g1_2, beta1_2, g2_2, beta2_2),
    )
    wn_ts = [Wf[:, af : 2 * af].T for (Wf, *_rest) in conv_params]

    x, q = _embed(atom_fea, We.T, be.reshape(1, -1), wn_ts[0])

    for li, (Wf, bf, g1, b1, g2, b2) in enumerate(conv_params):
        ws_t = Wf[:, :af].T
        we_t = Wf[:, 2 * af :].T
        bf2_ = bf.reshape(1, -1)
        qg = _gather_rows(q, idx_flat)
        stats = _conv_stats(x, qg, nf2, ws_t, we_t, bf2_)
        ns, st2 = _conv_apply(
            x, qg, nf2, ws_t, we_t, bf2_, stats,
            g1.reshape(1, -1), b1.reshape(1, -1), n * m,
        )
        if li < 2:
            x, q = _bn2_residual(
                x, ns, st2, g2.reshape(1, -1), b2.reshape(1, -1), wn_ts[li + 1]
            )
        else:
            out = _bn2_residual_head(
                x, ns, st2, g2.reshape(1, -1), b2.reshape(1, -1),
                Wc.T, bc.reshape(1, -1), Wo.T, bo.reshape(1, -1), n0, a,
            )
    return out
